# Initial kernel scaffold; baseline (speedup 1.0000x reference)
import jax
import jax.numpy as jnp
from jax.experimental import pallas as pl


def kernel(x, W_E, b_E, W_D, b_D):
    raise NotImplementedError("write your pallas kernel here")


# speedup vs baseline: 1.6033x; 1.6033x over previous
import functools

import jax
import jax.numpy as jnp
from jax.experimental import pallas as pl
from jax.experimental.pallas import tpu as pltpu

D_MODEL_ = 4096
D_SAE_ = 32768
TOP_K_ = 64
N_TOK_ = 32
TS_ = 1024
N_TILES_ = D_SAE_ // TS_


def _encode_body(x_ref, we_ref, be_ref, lat_ref, scale_ref):
    i = pl.program_id(0)

    @pl.when(i == 0)
    def _():
        x = x_ref[...]
        norms = jnp.sqrt(jnp.sum(x * x, axis=1, keepdims=True))
        scale_ref[...] = jnp.sqrt(jnp.float32(D_MODEL_)) / (norms + 1e-8)

    xn = x_ref[...] * scale_ref[...]
    pre = jnp.dot(xn, we_ref[...], preferred_element_type=jnp.float32)
    pre = pre + be_ref[...]
    lat_ref[...] = jnp.maximum(pre, 0.0)


def _topk_body(lat_ref, out_ref):
    lat = lat_ref[...]
    bits = jax.lax.bitcast_convert_type(lat, jnp.int32)
    lo = jnp.zeros((N_TOK_, 1), jnp.int32)
    hi = jnp.full((N_TOK_, 1), jnp.int32(0x7F000000), jnp.int32)

    def step(_, carry):
        lo, hi = carry
        mid = lo + (hi - lo + 1) // 2
        cnt = jnp.sum((bits >= mid).astype(jnp.int32), axis=1, keepdims=True)
        ge = cnt >= TOP_K_
        return jnp.where(ge, mid, lo), jnp.where(ge, hi, mid - 1)

    lo, hi = jax.lax.fori_loop(0, 31, step, (lo, hi))
    out_ref[...] = jnp.where(bits >= lo, lat, 0.0)


def _decode_body(lat_ref, wd_ref, bd_ref, x_ref, out_ref):
    i = pl.program_id(0)

    @pl.when(i == 0)
    def _():
        out_ref[...] = jnp.broadcast_to(bd_ref[...], (N_TOK_, D_MODEL_))

    out_ref[...] += jnp.dot(lat_ref[...], wd_ref[...],
                            preferred_element_type=jnp.float32)

    @pl.when(i == N_TILES_ - 1)
    def _():
        x = x_ref[...]
        norms = jnp.sqrt(jnp.sum(x * x, axis=1, keepdims=True))
        scale = jnp.sqrt(jnp.float32(D_MODEL_)) / (norms + 1e-8)
        out_ref[...] = out_ref[...] / scale


@jax.jit
def kernel(x, W_E, b_E, W_D, b_D):
    b_E2 = b_E.reshape(1, D_SAE_)
    b_D2 = b_D.reshape(1, D_MODEL_)

    latents = pl.pallas_call(
        _encode_body,
        grid=(N_TILES_,),
        in_specs=[
            pl.BlockSpec((N_TOK_, D_MODEL_), lambda i: (0, 0)),
            pl.BlockSpec((D_MODEL_, TS_), lambda i: (0, i)),
            pl.BlockSpec((1, TS_), lambda i: (0, i)),
        ],
        out_specs=pl.BlockSpec((N_TOK_, TS_), lambda i: (0, i)),
        out_shape=jax.ShapeDtypeStruct((N_TOK_, D_SAE_), jnp.float32),
        scratch_shapes=[pltpu.VMEM((N_TOK_, 1), jnp.float32)],
        compiler_params=pltpu.CompilerParams(
            dimension_semantics=("arbitrary",)),
    )(x, W_E, b_E2)

    latents_masked = pl.pallas_call(
        _topk_body,
        out_shape=jax.ShapeDtypeStruct((N_TOK_, D_SAE_), jnp.float32),
    )(latents)

    recon = pl.pallas_call(
        _decode_body,
        grid=(N_TILES_,),
        in_specs=[
            pl.BlockSpec((N_TOK_, TS_), lambda i: (0, i)),
            pl.BlockSpec((TS_, D_MODEL_), lambda i: (i, 0)),
            pl.BlockSpec((1, D_MODEL_), lambda i: (0, 0)),
            pl.BlockSpec((N_TOK_, D_MODEL_), lambda i: (0, 0)),
        ],
        out_specs=pl.BlockSpec((N_TOK_, D_MODEL_), lambda i: (0, 0)),
        out_shape=jax.ShapeDtypeStruct((N_TOK_, D_MODEL_), jnp.float32),
        compiler_params=pltpu.CompilerParams(
            dimension_semantics=("arbitrary",)),
    )(latents_masked, W_D, b_D2, x)

    return recon, latents_masked


# speedup vs baseline: 1.9348x; 1.2068x over previous
import functools

import jax
import jax.numpy as jnp
from jax import lax
from jax.experimental import pallas as pl
from jax.experimental.pallas import tpu as pltpu
from jax.experimental.pallas import tpu_sc as plsc

D_MODEL_ = 4096
D_SAE_ = 32768
TOP_K_ = 64
N_TOK_ = 32
TS_ = 1024
N_TILES_ = D_SAE_ // TS_

N_REGIONS_ = D_SAE_ // 128
CAND_ = 80
GROUP_ = 16


def _encode_body(x_ref, we_ref, be_ref, lat_ref, scale_ref, scale_scr):
    i = pl.program_id(0)

    @pl.when(i == 0)
    def _():
        x = x_ref[...]
        norms = jnp.sqrt(jnp.sum(x * x, axis=1, keepdims=True))
        scale = jnp.sqrt(jnp.float32(D_MODEL_)) / (norms + 1e-8)
        scale_scr[...] = scale
        scale_ref[...] = jnp.broadcast_to(scale, (N_TOK_, 128))

    xn = x_ref[...] * scale_scr[...]
    pre = jnp.dot(xn, we_ref[...], preferred_element_type=jnp.float32)
    pre = pre + be_ref[...]
    lat_ref[...] = jnp.maximum(pre, 0.0)


def _topk_body(lat_ref, out_ref):
    lat = lat_ref[...]
    bits = jax.lax.bitcast_convert_type(lat, jnp.int32)
    lo = jnp.zeros((N_TOK_, 1), jnp.int32)
    hi = jnp.full((N_TOK_, 1), jnp.int32(0x7F000000), jnp.int32)

    def step(_, carry):
        lo, hi = carry
        mid = lo + (hi - lo + 1) // 2
        cnt = jnp.sum((bits >= mid).astype(jnp.int32), axis=1, keepdims=True)
        ge = cnt >= TOP_K_
        return jnp.where(ge, mid, lo), jnp.where(ge, hi, mid - 1)

    lo, hi = jax.lax.fori_loop(0, 31, step, (lo, hi))
    out_ref[...] = jnp.where(bits >= lo, lat, 0.0)


def _sc_decode_body(lat_hbm, wd_hbm, bd_hbm, scale_hbm, out_hbm,
                    row_v, idx_v, val_v, rows_v, acc_v, bd_v,
                    scale_v, cnt_s, sem):
    nc = 2
    wid = lax.axis_index("s") * nc + lax.axis_index("c")

    pltpu.sync_copy(lat_hbm.at[wid], row_v)
    pltpu.sync_copy(bd_hbm, bd_v)
    pltpu.sync_copy(scale_hbm, scale_v)

    zero16f = jnp.zeros((16,), jnp.float32)
    zero16i = jnp.zeros((16,), jnp.int32)
    for c in range(CAND_ // 16):
        idx_v[pl.ds(c * 16, 16)] = zero16i
        val_v[pl.ds(c * 16, 16)] = zero16f
    cnt_s[0] = 0

    lane_iota = lax.iota(jnp.int32, 16)

    def chunk_body(c, carry):
        col0 = c * 16
        v = row_v[pl.ds(col0, 16)]
        m = v > 0.0
        k = plsc.all_reduce_population_count(m)[0]

        def _emit():
            cnt = cnt_s[0]
            off = jnp.minimum(cnt, CAND_ - 16)
            plsc.store_compressed(
                idx_v.at[pl.ds(off, 16)], lane_iota + col0, mask=m)
            plsc.store_compressed(val_v.at[pl.ds(off, 16)], v, mask=m)
            cnt_s[0] = cnt + k

        pl.when(k > 0)(_emit)
        return carry

    lax.fori_loop(0, D_SAE_ // 16, chunk_body, None)

    def zacc(j, carry):
        acc_v[pl.ds(j * 16, 16)] = zero16f
        return carry

    lax.fori_loop(0, D_MODEL_ // 16, zacc, None)

    for g in range(TOP_K_ // GROUP_):
        pltpu.async_copy(
            wd_hbm.at[idx_v.at[pl.ds(g * GROUP_, GROUP_)]], rows_v, sem
        ).wait()
        vv = val_v[pl.ds(g * GROUP_, GROUP_)]

        def fma(j, carry, vv=vv):
            col = j * 16
            a = acc_v[pl.ds(col, 16)]
            for rr in range(GROUP_):
                a = a + vv[rr] * rows_v[rr, pl.ds(col, 16)]
            acc_v[pl.ds(col, 16)] = a
            return carry

        lax.fori_loop(0, D_MODEL_ // 16, fma, None)

    sg = plsc.load_gather(scale_v, [jnp.full((16,), wid, jnp.int32)])

    def fin(j, carry):
        col = j * 16
        acc_v[pl.ds(col, 16)] = (
            acc_v[pl.ds(col, 16)] + bd_v[pl.ds(col, 16)]) / sg
        return carry

    lax.fori_loop(0, D_MODEL_ // 16, fin, None)
    pltpu.sync_copy(acc_v, out_hbm.at[wid])


_sc_decode = functools.partial(
    pl.kernel,
    out_type=jax.ShapeDtypeStruct((N_TOK_, D_MODEL_), jnp.float32),
    mesh=plsc.VectorSubcoreMesh(
        core_axis_name="c", subcore_axis_name="s", num_cores=2,
        num_subcores=16),
    scratch_types=[
        pltpu.VMEM((D_SAE_,), jnp.float32),
        pltpu.VMEM((CAND_,), jnp.int32),
        pltpu.VMEM((CAND_,), jnp.float32),
        pltpu.VMEM((GROUP_, D_MODEL_), jnp.float32),
        pltpu.VMEM((D_MODEL_,), jnp.float32),
        pltpu.VMEM((D_MODEL_,), jnp.float32),
        pltpu.VMEM((N_TOK_,), jnp.float32),
        pltpu.SMEM((1,), jnp.int32),
        pltpu.SemaphoreType.DMA,
    ],
    compiler_params=pltpu.CompilerParams(needs_layout_passes=False),
)(_sc_decode_body)


@jax.jit
def kernel(x, W_E, b_E, W_D, b_D):
    b_E2 = b_E.reshape(1, D_SAE_)

    latents, scale_blk = pl.pallas_call(
        _encode_body,
        grid=(N_TILES_,),
        in_specs=[
            pl.BlockSpec((N_TOK_, D_MODEL_), lambda i: (0, 0)),
            pl.BlockSpec((D_MODEL_, TS_), lambda i: (0, i)),
            pl.BlockSpec((1, TS_), lambda i: (0, i)),
        ],
        out_specs=[
            pl.BlockSpec((N_TOK_, TS_), lambda i: (0, i)),
            pl.BlockSpec((N_TOK_, 128), lambda i: (0, 0)),
        ],
        out_shape=[
            jax.ShapeDtypeStruct((N_TOK_, D_SAE_), jnp.float32),
            jax.ShapeDtypeStruct((N_TOK_, 128), jnp.float32),
        ],
        scratch_shapes=[pltpu.VMEM((N_TOK_, 1), jnp.float32)],
        compiler_params=pltpu.CompilerParams(
            dimension_semantics=("arbitrary",)),
    )(x, W_E, b_E2)

    latents_masked = pl.pallas_call(
        _topk_body,
        out_shape=jax.ShapeDtypeStruct((N_TOK_, D_SAE_), jnp.float32),
    )(latents)

    scale_vec = scale_blk[:, 0]
    recon = _sc_decode(latents_masked, W_D, b_D, scale_vec)
    return recon, latents_masked


# speedup vs baseline: 1.9914x; 1.0293x over previous
import functools

import jax
import jax.numpy as jnp
from jax import lax
from jax.experimental import pallas as pl
from jax.experimental.pallas import tpu as pltpu
from jax.experimental.pallas import tpu_sc as plsc

D_MODEL_ = 4096
D_SAE_ = 32768
TOP_K_ = 64
N_TOK_ = 32
TS_ = 1024
N_TILES_ = D_SAE_ // TS_

N_REGIONS_ = D_SAE_ // 128
CAND_ = 80
GROUP_ = 8


def _encode_body(x_ref, we_ref, be_ref, lat_ref, scale_ref, scale_scr):
    i = pl.program_id(0)

    @pl.when(i == 0)
    def _():
        x = x_ref[...]
        norms = jnp.sqrt(jnp.sum(x * x, axis=1, keepdims=True))
        scale = jnp.sqrt(jnp.float32(D_MODEL_)) / (norms + 1e-8)
        scale_scr[...] = scale
        scale_ref[...] = jnp.broadcast_to(scale, (N_TOK_, 128))

    xn = x_ref[...] * scale_scr[...]
    pre = jnp.dot(xn, we_ref[...], preferred_element_type=jnp.float32)
    pre = pre + be_ref[...]
    lat_ref[...] = jnp.maximum(pre, 0.0)


def _topk_body(lat_ref, out_ref):
    lat = lat_ref[...]
    bits = jax.lax.bitcast_convert_type(lat, jnp.int32)
    lo0 = jnp.zeros((N_TOK_, 1), jnp.int32)
    hi0 = jnp.full((N_TOK_, 1), jnp.int32(0x7F000000), jnp.int32)
    thr0 = jnp.zeros((N_TOK_, 1), jnp.int32)
    found0 = jnp.zeros((N_TOK_, 1), jnp.int32)

    def cond(c):
        _, _, _, found = c
        return jnp.min(found) < 1

    def body(c):
        lo, hi, thr, found = c
        mid = lo + (hi - lo + 1) // 2
        cnt = jnp.sum((bits >= mid).astype(jnp.int32), axis=1, keepdims=True)
        live = found < 1
        hit = jnp.logical_and(cnt == TOP_K_, live)
        thr = jnp.where(hit, mid, thr)
        found2 = jnp.where(hit, 1, found)
        ge = cnt >= TOP_K_
        lo = jnp.where(live, jnp.where(ge, mid, lo), lo)
        hi = jnp.where(live, jnp.where(ge, hi, mid - 1), hi)
        conv = jnp.logical_and(lo >= hi, found2 < 1)
        thr = jnp.where(conv, lo, thr)
        found2 = jnp.where(conv, 1, found2)
        return lo, hi, thr, found2

    _, _, thr, _ = jax.lax.while_loop(cond, body, (lo0, hi0, thr0, found0))
    out_ref[...] = jnp.where(bits >= thr, lat, 0.0)


def _sc_decode_body(lat_hbm, wd_hbm, bd_hbm, scale_hbm, out_hbm,
                    row_v, idx_v, val_v, rows0_v, rows1_v, acc_v, bd_v,
                    scale_v, cnt_s, sem, sem1):
    nc = 2
    wid = lax.axis_index("s") * nc + lax.axis_index("c")

    pltpu.sync_copy(lat_hbm.at[wid], row_v)
    pltpu.sync_copy(bd_hbm, bd_v)
    pltpu.sync_copy(scale_hbm, scale_v)

    zero16f = jnp.zeros((16,), jnp.float32)
    zero16i = jnp.zeros((16,), jnp.int32)
    for c in range(CAND_ // 16):
        idx_v[pl.ds(c * 16, 16)] = zero16i
        val_v[pl.ds(c * 16, 16)] = zero16f
    cnt_s[0] = 0

    lane_iota = lax.iota(jnp.int32, 16)

    def chunk_body(c, carry):
        col0 = c * 16
        v = row_v[pl.ds(col0, 16)]
        m = v > 0.0
        k = plsc.all_reduce_population_count(m)[0]

        def _emit():
            cnt = cnt_s[0]
            off = jnp.minimum(cnt, CAND_ - 16)
            plsc.store_compressed(
                idx_v.at[pl.ds(off, 16)], lane_iota + col0, mask=m)
            plsc.store_compressed(val_v.at[pl.ds(off, 16)], v, mask=m)
            cnt_s[0] = cnt + k

        pl.when(k > 0)(_emit)
        return carry

    lax.fori_loop(0, D_SAE_ // 16, chunk_body, None)

    def zacc(j, carry):
        acc_v[pl.ds(j * 16, 16)] = zero16f
        return carry

    lax.fori_loop(0, D_MODEL_ // 16, zacc, None)

    n_rounds = TOP_K_ // GROUP_
    bufs = (rows0_v, rows1_v)
    sems = (sem, sem1)

    def start_gather(g):
        return pltpu.async_copy(
            wd_hbm.at[idx_v.at[pl.ds(g * GROUP_, GROUP_)]],
            bufs[g % 2], sems[g % 2])

    cp = start_gather(0)
    vv = None
    for g in range(n_rounds):
        if g % 2 == 0:
            vv = val_v[pl.ds(g * GROUP_, 16)]
        cp.wait()
        if g + 1 < n_rounds:
            cp = start_gather(g + 1)
        rows = bufs[g % 2]
        lane0 = (g % 2) * GROUP_

        def fma(j, carry, vv=vv, rows=rows, lane0=lane0):
            col = j * 16
            a = acc_v[pl.ds(col, 16)]
            for rr in range(GROUP_):
                a = a + vv[lane0 + rr] * rows[rr, pl.ds(col, 16)]
            acc_v[pl.ds(col, 16)] = a
            return carry

        lax.fori_loop(0, D_MODEL_ // 16, fma, None)

    sg = plsc.load_gather(scale_v, [jnp.full((16,), wid, jnp.int32)])

    def fin(j, carry):
        col = j * 16
        acc_v[pl.ds(col, 16)] = (
            acc_v[pl.ds(col, 16)] + bd_v[pl.ds(col, 16)]) / sg
        return carry

    lax.fori_loop(0, D_MODEL_ // 16, fin, None)
    pltpu.sync_copy(acc_v, out_hbm.at[wid])


_sc_decode = functools.partial(
    pl.kernel,
    out_type=jax.ShapeDtypeStruct((N_TOK_, D_MODEL_), jnp.float32),
    mesh=plsc.VectorSubcoreMesh(
        core_axis_name="c", subcore_axis_name="s", num_cores=2,
        num_subcores=16),
    scratch_types=[
        pltpu.VMEM((D_SAE_,), jnp.float32),
        pltpu.VMEM((CAND_,), jnp.int32),
        pltpu.VMEM((CAND_,), jnp.float32),
        pltpu.VMEM((GROUP_, D_MODEL_), jnp.float32),
        pltpu.VMEM((GROUP_, D_MODEL_), jnp.float32),
        pltpu.VMEM((D_MODEL_,), jnp.float32),
        pltpu.VMEM((D_MODEL_,), jnp.float32),
        pltpu.VMEM((N_TOK_,), jnp.float32),
        pltpu.SMEM((1,), jnp.int32),
        pltpu.SemaphoreType.DMA,
        pltpu.SemaphoreType.DMA,
    ],
    compiler_params=pltpu.CompilerParams(needs_layout_passes=False),
)(_sc_decode_body)


@jax.jit
def kernel(x, W_E, b_E, W_D, b_D):
    b_E2 = b_E.reshape(1, D_SAE_)

    latents, scale_blk = pl.pallas_call(
        _encode_body,
        grid=(N_TILES_,),
        in_specs=[
            pl.BlockSpec((N_TOK_, D_MODEL_), lambda i: (0, 0)),
            pl.BlockSpec((D_MODEL_, TS_), lambda i: (0, i)),
            pl.BlockSpec((1, TS_), lambda i: (0, i)),
        ],
        out_specs=[
            pl.BlockSpec((N_TOK_, TS_), lambda i: (0, i)),
            pl.BlockSpec((N_TOK_, 128), lambda i: (0, 0)),
        ],
        out_shape=[
            jax.ShapeDtypeStruct((N_TOK_, D_SAE_), jnp.float32),
            jax.ShapeDtypeStruct((N_TOK_, 128), jnp.float32),
        ],
        scratch_shapes=[pltpu.VMEM((N_TOK_, 1), jnp.float32)],
        compiler_params=pltpu.CompilerParams(
            dimension_semantics=("arbitrary",)),
    )(x, W_E, b_E2)

    latents_masked = pl.pallas_call(
        _topk_body,
        out_shape=jax.ShapeDtypeStruct((N_TOK_, D_SAE_), jnp.float32),
    )(latents)

    scale_vec = scale_blk[:, 0]
    recon = _sc_decode(latents_masked, W_D, b_D, scale_vec)
    return recon, latents_masked


# speedup vs baseline: 2.1458x; 1.0775x over previous
import functools

import jax
import jax.numpy as jnp
from jax import lax
from jax.experimental import pallas as pl
from jax.experimental.pallas import tpu as pltpu
from jax.experimental.pallas import tpu_sc as plsc

D_MODEL_ = 4096
D_SAE_ = 32768
TOP_K_ = 64
N_TOK_ = 32
TS_ = 1024
N_TILES_ = D_SAE_ // TS_

N_REGIONS_ = D_SAE_ // 128
CAND_ = 80
GROUP_ = 8


def _encode_body(x_ref, we_ref, be_ref, lat_ref, scale_ref, scale_scr):
    i = pl.program_id(0)

    @pl.when(i == 0)
    def _():
        x = x_ref[...]
        norms = jnp.sqrt(jnp.sum(x * x, axis=1, keepdims=True))
        scale = jnp.sqrt(jnp.float32(D_MODEL_)) / (norms + 1e-8)
        scale_scr[...] = scale
        scale_ref[...] = jnp.broadcast_to(scale, (N_TOK_, 128))

    xn = x_ref[...] * scale_scr[...]
    pre = jnp.dot(xn, we_ref[...], preferred_element_type=jnp.float32)
    pre = pre + be_ref[...]
    lat_ref[...] = jnp.maximum(pre, 0.0)


def _topk_body(lat_ref, out_ref, flag_ref):
    lat = lat_ref[...]
    bits = jax.lax.bitcast_convert_type(lat, jnp.int32)
    lo0 = jnp.zeros((N_TOK_, 1), jnp.int32)
    hi0 = jnp.full((N_TOK_, 1), jnp.int32(0x7F000000), jnp.int32)
    thr0 = jnp.zeros((N_TOK_, 1), jnp.int32)
    found0 = jnp.zeros((N_TOK_, 1), jnp.int32)

    def cond(c):
        _, _, _, found = c
        return jnp.min(found) < 1

    def body(c):
        lo, hi, thr, found = c
        mid = lo + (hi - lo + 1) // 2
        cnt = jnp.sum((bits >= mid).astype(jnp.int32), axis=1, keepdims=True)
        live = found < 1
        hit = jnp.logical_and(cnt == TOP_K_, live)
        thr = jnp.where(hit, mid, thr)
        found2 = jnp.where(hit, 1, found)
        ge = cnt >= TOP_K_
        lo = jnp.where(live, jnp.where(ge, mid, lo), lo)
        hi = jnp.where(live, jnp.where(ge, hi, mid - 1), hi)
        conv = jnp.logical_and(lo >= hi, found2 < 1)
        thr = jnp.where(conv, lo, thr)
        found2 = jnp.where(conv, 1, found2)
        return lo, hi, thr, found2

    _, _, thr, _ = jax.lax.while_loop(cond, body, (lo0, hi0, thr0, found0))
    masked = jnp.where(bits >= thr, lat, 0.0)
    out_ref[...] = masked
    flag_ref[...] = jnp.max(
        masked.reshape(N_TOK_, N_REGIONS_, 128), axis=-1)


def _sc_decode_body(lat_hbm, flag_hbm, wd_hbm, bd_hbm, scale_hbm, out_hbm,
                    row_v, flags_v, idx_v, val_v, rows0_v, rows1_v, acc_v,
                    bd_v, scale_v, cnt_s, sem, sem1):
    nc = 2
    wid = lax.axis_index("s") * nc + lax.axis_index("c")

    pltpu.sync_copy(lat_hbm.at[wid], row_v)
    pltpu.sync_copy(flag_hbm.at[wid], flags_v)
    pltpu.sync_copy(bd_hbm, bd_v)
    pltpu.sync_copy(scale_hbm, scale_v)

    zero16f = jnp.zeros((16,), jnp.float32)
    zero16i = jnp.zeros((16,), jnp.int32)
    for c in range(CAND_ // 16):
        idx_v[pl.ds(c * 16, 16)] = zero16i
        val_v[pl.ds(c * 16, 16)] = zero16f
    cnt_s[0] = 0

    lane_iota = lax.iota(jnp.int32, 16)

    def region_body(r, carry):
        fr = plsc.load_gather(flags_v, [jnp.full((16,), r, jnp.int32)])

        def _drill():
            base = r * 128
            for cc in range(8):
                v = row_v[pl.ds(base + cc * 16, 16)]
                m = v > 0.0
                k = plsc.all_reduce_population_count(m)[0]

                def _emit(v=v, m=m, k=k, c0=cc * 16):
                    cnt = cnt_s[0]
                    off = jnp.minimum(cnt, CAND_ - 16)
                    plsc.store_compressed(
                        idx_v.at[pl.ds(off, 16)],
                        lane_iota + (base + c0), mask=m)
                    plsc.store_compressed(
                        val_v.at[pl.ds(off, 16)], v, mask=m)
                    cnt_s[0] = cnt + k

                pl.when(k > 0)(_emit)

        pl.when(fr[0] > 0.0)(_drill)
        return carry

    lax.fori_loop(0, N_REGIONS_, region_body, None)

    def zacc(j, carry):
        acc_v[pl.ds(j * 16, 16)] = zero16f
        return carry

    lax.fori_loop(0, D_MODEL_ // 16, zacc, None)

    n_rounds = TOP_K_ // GROUP_
    bufs = (rows0_v, rows1_v)
    sems = (sem, sem1)

    def start_gather(g):
        return pltpu.async_copy(
            wd_hbm.at[idx_v.at[pl.ds(g * GROUP_, GROUP_)]],
            bufs[g % 2], sems[g % 2])

    cp = start_gather(0)
    vv = None
    for g in range(n_rounds):
        if g % 2 == 0:
            vv = val_v[pl.ds(g * GROUP_, 16)]
        cp.wait()
        if g + 1 < n_rounds:
            cp = start_gather(g + 1)
        rows = bufs[g % 2]
        lane0 = (g % 2) * GROUP_

        def fma(j, carry, vv=vv, rows=rows, lane0=lane0):
            col = j * 16
            a = acc_v[pl.ds(col, 16)]
            for rr in range(GROUP_):
                a = a + vv[lane0 + rr] * rows[rr, pl.ds(col, 16)]
            acc_v[pl.ds(col, 16)] = a
            return carry

        lax.fori_loop(0, D_MODEL_ // 16, fma, None)

    sg = plsc.load_gather(scale_v, [jnp.full((16,), wid, jnp.int32)])

    def fin(j, carry):
        col = j * 16
        acc_v[pl.ds(col, 16)] = (
            acc_v[pl.ds(col, 16)] + bd_v[pl.ds(col, 16)]) / sg
        return carry

    lax.fori_loop(0, D_MODEL_ // 16, fin, None)
    pltpu.sync_copy(acc_v, out_hbm.at[wid])


_sc_decode = functools.partial(
    pl.kernel,
    out_type=jax.ShapeDtypeStruct((N_TOK_, D_MODEL_), jnp.float32),
    mesh=plsc.VectorSubcoreMesh(
        core_axis_name="c", subcore_axis_name="s", num_cores=2,
        num_subcores=16),
    scratch_types=[
        pltpu.VMEM((D_SAE_,), jnp.float32),
        pltpu.VMEM((N_REGIONS_,), jnp.float32),
        pltpu.VMEM((CAND_,), jnp.int32),
        pltpu.VMEM((CAND_,), jnp.float32),
        pltpu.VMEM((GROUP_, D_MODEL_), jnp.float32),
        pltpu.VMEM((GROUP_, D_MODEL_), jnp.float32),
        pltpu.VMEM((D_MODEL_,), jnp.float32),
        pltpu.VMEM((D_MODEL_,), jnp.float32),
        pltpu.VMEM((N_TOK_,), jnp.float32),
        pltpu.SMEM((1,), jnp.int32),
        pltpu.SemaphoreType.DMA,
        pltpu.SemaphoreType.DMA,
    ],
    compiler_params=pltpu.CompilerParams(needs_layout_passes=False),
)(_sc_decode_body)


@jax.jit
def kernel(x, W_E, b_E, W_D, b_D):
    b_E2 = b_E.reshape(1, D_SAE_)

    latents, scale_blk = pl.pallas_call(
        _encode_body,
        grid=(N_TILES_,),
        in_specs=[
            pl.BlockSpec((N_TOK_, D_MODEL_), lambda i: (0, 0)),
            pl.BlockSpec((D_MODEL_, TS_), lambda i: (0, i)),
            pl.BlockSpec((1, TS_), lambda i: (0, i)),
        ],
        out_specs=[
            pl.BlockSpec((N_TOK_, TS_), lambda i: (0, i)),
            pl.BlockSpec((N_TOK_, 128), lambda i: (0, 0)),
        ],
        out_shape=[
            jax.ShapeDtypeStruct((N_TOK_, D_SAE_), jnp.float32),
            jax.ShapeDtypeStruct((N_TOK_, 128), jnp.float32),
        ],
        scratch_shapes=[pltpu.VMEM((N_TOK_, 1), jnp.float32)],
        compiler_params=pltpu.CompilerParams(
            dimension_semantics=("arbitrary",)),
    )(x, W_E, b_E2)

    latents_masked, flags = pl.pallas_call(
        _topk_body,
        out_shape=[
            jax.ShapeDtypeStruct((N_TOK_, D_SAE_), jnp.float32),
            jax.ShapeDtypeStruct((N_TOK_, N_REGIONS_), jnp.float32),
        ],
    )(latents)

    scale_vec = scale_blk[:, 0]
    recon = _sc_decode(latents_masked, flags, W_D, b_D, scale_vec)
    return recon, latents_masked


# speedup vs baseline: 2.1636x; 1.0083x over previous
import functools

import jax
import jax.numpy as jnp
from jax import lax
from jax.experimental import pallas as pl
from jax.experimental.pallas import tpu as pltpu
from jax.experimental.pallas import tpu_sc as plsc

D_MODEL_ = 4096
D_SAE_ = 32768
TOP_K_ = 64
N_TOK_ = 32
TS_ = 1024
N_TILES_ = D_SAE_ // TS_

N_REGIONS_ = D_SAE_ // 128
CAND_ = 80
GROUP_ = 8


def _encode_body(x_ref, we_ref, be_ref, lat_ref, scale_ref, scale_scr):
    i = pl.program_id(0)

    @pl.when(i == 0)
    def _():
        x = x_ref[...]
        norms = jnp.sqrt(jnp.sum(x * x, axis=1, keepdims=True))
        scale = jnp.sqrt(jnp.float32(D_MODEL_)) / (norms + 1e-8)
        scale_scr[...] = scale
        scale_ref[...] = jnp.broadcast_to(scale, (N_TOK_, 128))

    xn = x_ref[...] * scale_scr[...]
    pre = jnp.dot(xn, we_ref[...], preferred_element_type=jnp.float32)
    pre = pre + be_ref[...]
    lat_ref[...] = jnp.maximum(pre, 0.0)


def _topk_body(lat_ref, out_ref, flag_ref):
    lat = lat_ref[...]
    bits = jax.lax.bitcast_convert_type(lat, jnp.int32)
    lo0 = jnp.zeros((N_TOK_, 1), jnp.int32)
    hi0 = jnp.full((N_TOK_, 1), jnp.int32(0x7F000000), jnp.int32)
    thr0 = jnp.zeros((N_TOK_, 1), jnp.int32)
    found0 = jnp.zeros((N_TOK_, 1), jnp.int32)

    def cond(c):
        _, _, _, found = c
        return jnp.min(found) < 1

    def body(c):
        lo, hi, thr, found = c
        mid = lo + (hi - lo + 1) // 2
        cnt = jnp.sum((bits >= mid).astype(jnp.int32), axis=1, keepdims=True)
        live = found < 1
        hit = jnp.logical_and(cnt == TOP_K_, live)
        thr = jnp.where(hit, mid, thr)
        found2 = jnp.where(hit, 1, found)
        ge = cnt >= TOP_K_
        lo = jnp.where(live, jnp.where(ge, mid, lo), lo)
        hi = jnp.where(live, jnp.where(ge, hi, mid - 1), hi)
        conv = jnp.logical_and(lo >= hi, found2 < 1)
        thr = jnp.where(conv, lo, thr)
        found2 = jnp.where(conv, 1, found2)
        return lo, hi, thr, found2

    _, _, thr, _ = jax.lax.while_loop(cond, body, (lo0, hi0, thr0, found0))
    masked = jnp.where(bits >= thr, lat, 0.0)
    out_ref[...] = masked
    flag_ref[...] = jnp.max(
        masked.reshape(N_TOK_, N_REGIONS_, 128), axis=-1)


def _sc_decode_body(lat_hbm, flag_hbm, wd_hbm, bd_hbm, scale_hbm, out_hbm,
                    row_v, flags_v, idx_v, val_v, rows0_v, rows1_v, acc_v,
                    bd_v, scale_v, cnt_s, sem, sem1):
    nc = 2
    wid = lax.axis_index("s") * nc + lax.axis_index("c")

    pltpu.sync_copy(lat_hbm.at[wid], row_v)
    pltpu.sync_copy(flag_hbm.at[wid], flags_v)
    pltpu.sync_copy(bd_hbm, bd_v)
    pltpu.sync_copy(scale_hbm, scale_v)

    zero16f = jnp.zeros((16,), jnp.float32)
    zero16i = jnp.zeros((16,), jnp.int32)
    for c in range(CAND_ // 16):
        idx_v[pl.ds(c * 16, 16)] = zero16i
        val_v[pl.ds(c * 16, 16)] = zero16f
    cnt_s[0] = 0

    lane_iota = lax.iota(jnp.int32, 16)

    def region_body(r, carry):
        fr = plsc.load_gather(flags_v, [jnp.full((16,), r, jnp.int32)])

        def _drill():
            base = r * 128
            for cc in range(8):
                v = row_v[pl.ds(base + cc * 16, 16)]
                m = v > 0.0
                k = plsc.all_reduce_population_count(m)[0]

                def _emit(v=v, m=m, k=k, c0=cc * 16):
                    cnt = cnt_s[0]
                    off = jnp.minimum(cnt, CAND_ - 16)
                    plsc.store_compressed(
                        idx_v.at[pl.ds(off, 16)],
                        lane_iota + (base + c0), mask=m)
                    plsc.store_compressed(
                        val_v.at[pl.ds(off, 16)], v, mask=m)
                    cnt_s[0] = cnt + k

                pl.when(k > 0)(_emit)

        pl.when(fr[0] > 0.0)(_drill)
        return carry

    lax.fori_loop(0, N_REGIONS_, region_body, None)

    n_rounds = TOP_K_ // GROUP_
    bufs = (rows0_v, rows1_v)
    sems = (sem, sem1)

    def start_gather(g):
        return pltpu.async_copy(
            wd_hbm.at[idx_v.at[pl.ds(g * GROUP_, GROUP_)]],
            bufs[g % 2], sems[g % 2])

    sg = plsc.load_gather(scale_v, [jnp.full((16,), wid, jnp.int32)])
    cp = start_gather(0)
    vv = None
    for g in range(n_rounds):
        if g % 2 == 0:
            vv = val_v[pl.ds(g * GROUP_, 16)]
        cp.wait()
        if g + 1 < n_rounds:
            cp = start_gather(g + 1)
        rows = bufs[g % 2]
        lane0 = (g % 2) * GROUP_

        first, last = g == 0, g == n_rounds - 1

        def fma(j, carry, vv=vv, rows=rows, lane0=lane0,
                first=first, last=last):
            col = j * 16
            s = pl.ds(col, 16)
            a = vv[lane0] * rows[0, s]
            if not first:
                a = a + acc_v[s]
            for rr in range(1, GROUP_):
                a = a + vv[lane0 + rr] * rows[rr, s]
            if last:
                a = (a + bd_v[s]) / sg
            acc_v[s] = a
            return carry

        lax.fori_loop(0, D_MODEL_ // 16, fma, None)

    pltpu.sync_copy(acc_v, out_hbm.at[wid])


_sc_decode = functools.partial(
    pl.kernel,
    out_type=jax.ShapeDtypeStruct((N_TOK_, D_MODEL_), jnp.float32),
    mesh=plsc.VectorSubcoreMesh(
        core_axis_name="c", subcore_axis_name="s", num_cores=2,
        num_subcores=16),
    scratch_types=[
        pltpu.VMEM((D_SAE_,), jnp.float32),
        pltpu.VMEM((N_REGIONS_,), jnp.float32),
        pltpu.VMEM((CAND_,), jnp.int32),
        pltpu.VMEM((CAND_,), jnp.float32),
        pltpu.VMEM((GROUP_, D_MODEL_), jnp.float32),
        pltpu.VMEM((GROUP_, D_MODEL_), jnp.float32),
        pltpu.VMEM((D_MODEL_,), jnp.float32),
        pltpu.VMEM((D_MODEL_,), jnp.float32),
        pltpu.VMEM((N_TOK_,), jnp.float32),
        pltpu.SMEM((1,), jnp.int32),
        pltpu.SemaphoreType.DMA,
        pltpu.SemaphoreType.DMA,
    ],
    compiler_params=pltpu.CompilerParams(needs_layout_passes=False),
)(_sc_decode_body)


@jax.jit
def kernel(x, W_E, b_E, W_D, b_D):
    b_E2 = b_E.reshape(1, D_SAE_)

    latents, scale_blk = pl.pallas_call(
        _encode_body,
        grid=(N_TILES_,),
        in_specs=[
            pl.BlockSpec((N_TOK_, D_MODEL_), lambda i: (0, 0)),
            pl.BlockSpec((D_MODEL_, TS_), lambda i: (0, i)),
            pl.BlockSpec((1, TS_), lambda i: (0, i)),
        ],
        out_specs=[
            pl.BlockSpec((N_TOK_, TS_), lambda i: (0, i)),
            pl.BlockSpec((N_TOK_, 128), lambda i: (0, 0)),
        ],
        out_shape=[
            jax.ShapeDtypeStruct((N_TOK_, D_SAE_), jnp.float32),
            jax.ShapeDtypeStruct((N_TOK_, 128), jnp.float32),
        ],
        scratch_shapes=[pltpu.VMEM((N_TOK_, 1), jnp.float32)],
        compiler_params=pltpu.CompilerParams(
            dimension_semantics=("arbitrary",)),
    )(x, W_E, b_E2)

    latents_masked, flags = pl.pallas_call(
        _topk_body,
        out_shape=[
            jax.ShapeDtypeStruct((N_TOK_, D_SAE_), jnp.float32),
            jax.ShapeDtypeStruct((N_TOK_, N_REGIONS_), jnp.float32),
        ],
    )(latents)

    scale_vec = scale_blk[:, 0]
    recon = _sc_decode(latents_masked, flags, W_D, b_D, scale_vec)
    return recon, latents_masked


# speedup vs baseline: 2.1845x; 1.0097x over previous
import functools

import jax
import jax.numpy as jnp
from jax import lax
from jax.experimental import pallas as pl
from jax.experimental.pallas import tpu as pltpu
from jax.experimental.pallas import tpu_sc as plsc

D_MODEL_ = 4096
D_SAE_ = 32768
TOP_K_ = 64
N_TOK_ = 32
TS_ = 1024
N_TILES_ = D_SAE_ // TS_

N_REGIONS_ = D_SAE_ // 128
CAND_ = 80
GROUP_ = 8


def _enc_topk_body(x_ref, we_ref, be_ref, out_ref, flag_ref, scale_ref,
                   lat_scr, scale_scr):
    i = pl.program_id(0)

    @pl.when(i == 0)
    def _():
        x = x_ref[...]
        norms = jnp.sqrt(jnp.sum(x * x, axis=1, keepdims=True))
        scale = jnp.sqrt(jnp.float32(D_MODEL_)) / (norms + 1e-8)
        scale_scr[...] = scale
        scale_ref[...] = jnp.broadcast_to(scale, (N_TOK_, 128))

    xn = x_ref[...] * scale_scr[...]
    pre = jnp.dot(xn, we_ref[...], preferred_element_type=jnp.float32)
    pre = pre + be_ref[...]
    lat_scr[:, pl.ds(i * TS_, TS_)] = jnp.maximum(pre, 0.0)

    @pl.when(i == N_TILES_ - 1)
    def _():
        lat = lat_scr[...]
        bits = jax.lax.bitcast_convert_type(lat, jnp.int32)
        lo0 = jnp.zeros((N_TOK_, 1), jnp.int32)
        hi0 = jnp.full((N_TOK_, 1), jnp.int32(0x7F000000), jnp.int32)
        thr0 = jnp.zeros((N_TOK_, 1), jnp.int32)
        found0 = jnp.zeros((N_TOK_, 1), jnp.int32)

        def cond(c):
            _, _, _, found = c
            return jnp.min(found) < 1

        def body(c):
            lo, hi, thr, found = c
            mid = lo + (hi - lo + 1) // 2
            cnt = jnp.sum((bits >= mid).astype(jnp.int32), axis=1,
                          keepdims=True)
            live = found < 1
            hit = jnp.logical_and(cnt == TOP_K_, live)
            thr = jnp.where(hit, mid, thr)
            found2 = jnp.where(hit, 1, found)
            ge = cnt >= TOP_K_
            lo = jnp.where(live, jnp.where(ge, mid, lo), lo)
            hi = jnp.where(live, jnp.where(ge, hi, mid - 1), hi)
            conv = jnp.logical_and(lo >= hi, found2 < 1)
            thr = jnp.where(conv, lo, thr)
            found2 = jnp.where(conv, 1, found2)
            return lo, hi, thr, found2

        _, _, thr, _ = jax.lax.while_loop(cond, body,
                                          (lo0, hi0, thr0, found0))
        masked = jnp.where(bits >= thr, lat, 0.0)
        out_ref[...] = masked
        flag_ref[...] = jnp.max(
            masked.reshape(N_TOK_, N_REGIONS_, 128), axis=-1)


def _sc_decode_body(lat_hbm, flag_hbm, wd_hbm, bd_hbm, scale_hbm, out_hbm,
                    row_v, flags_v, idx_v, val_v, rows0_v, rows1_v, acc_v,
                    bd_v, scale_v, cnt_s, sem, sem1):
    nc = 2
    wid = lax.axis_index("s") * nc + lax.axis_index("c")

    pltpu.sync_copy(lat_hbm.at[wid], row_v)
    pltpu.sync_copy(flag_hbm.at[wid], flags_v)
    pltpu.sync_copy(bd_hbm, bd_v)
    pltpu.sync_copy(scale_hbm, scale_v)

    zero16f = jnp.zeros((16,), jnp.float32)
    zero16i = jnp.zeros((16,), jnp.int32)
    for c in range(CAND_ // 16):
        idx_v[pl.ds(c * 16, 16)] = zero16i
        val_v[pl.ds(c * 16, 16)] = zero16f
    cnt_s[0] = 0

    lane_iota = lax.iota(jnp.int32, 16)

    def region_body(r, carry):
        fr = plsc.load_gather(flags_v, [jnp.full((16,), r, jnp.int32)])

        def _drill():
            base = r * 128
            for cc in range(8):
                v = row_v[pl.ds(base + cc * 16, 16)]
                m = v > 0.0
                k = plsc.all_reduce_population_count(m)[0]

                def _emit(v=v, m=m, k=k, c0=cc * 16):
                    cnt = cnt_s[0]
                    off = jnp.minimum(cnt, CAND_ - 16)
                    plsc.store_compressed(
                        idx_v.at[pl.ds(off, 16)],
                        lane_iota + (base + c0), mask=m)
                    plsc.store_compressed(
                        val_v.at[pl.ds(off, 16)], v, mask=m)
                    cnt_s[0] = cnt + k

                pl.when(k > 0)(_emit)

        pl.when(fr[0] > 0.0)(_drill)
        return carry

    lax.fori_loop(0, N_REGIONS_, region_body, None)

    n_rounds = TOP_K_ // GROUP_
    bufs = (rows0_v, rows1_v)
    sems = (sem, sem1)

    def start_gather(g):
        return pltpu.async_copy(
            wd_hbm.at[idx_v.at[pl.ds(g * GROUP_, GROUP_)]],
            bufs[g % 2], sems[g % 2])

    sg = plsc.load_gather(scale_v, [jnp.full((16,), wid, jnp.int32)])
    cp = start_gather(0)
    vv = None
    for g in range(n_rounds):
        if g % 2 == 0:
            vv = val_v[pl.ds(g * GROUP_, 16)]
        cp.wait()
        if g + 1 < n_rounds:
            cp = start_gather(g + 1)
        rows = bufs[g % 2]
        lane0 = (g % 2) * GROUP_

        first, last = g == 0, g == n_rounds - 1

        def fma(j, carry, vv=vv, rows=rows, lane0=lane0,
                first=first, last=last):
            col = j * 16
            s = pl.ds(col, 16)
            a = vv[lane0] * rows[0, s]
            if not first:
                a = a + acc_v[s]
            for rr in range(1, GROUP_):
                a = a + vv[lane0 + rr] * rows[rr, s]
            if last:
                a = (a + bd_v[s]) / sg
            acc_v[s] = a
            return carry

        lax.fori_loop(0, D_MODEL_ // 16, fma, None)

    pltpu.sync_copy(acc_v, out_hbm.at[wid])


_sc_decode = functools.partial(
    pl.kernel,
    out_type=jax.ShapeDtypeStruct((N_TOK_, D_MODEL_), jnp.float32),
    mesh=plsc.VectorSubcoreMesh(
        core_axis_name="c", subcore_axis_name="s", num_cores=2,
        num_subcores=16),
    scratch_types=[
        pltpu.VMEM((D_SAE_,), jnp.float32),
        pltpu.VMEM((N_REGIONS_,), jnp.float32),
        pltpu.VMEM((CAND_,), jnp.int32),
        pltpu.VMEM((CAND_,), jnp.float32),
        pltpu.VMEM((GROUP_, D_MODEL_), jnp.float32),
        pltpu.VMEM((GROUP_, D_MODEL_), jnp.float32),
        pltpu.VMEM((D_MODEL_,), jnp.float32),
        pltpu.VMEM((D_MODEL_,), jnp.float32),
        pltpu.VMEM((N_TOK_,), jnp.float32),
        pltpu.SMEM((1,), jnp.int32),
        pltpu.SemaphoreType.DMA,
        pltpu.SemaphoreType.DMA,
    ],
    compiler_params=pltpu.CompilerParams(needs_layout_passes=False),
)(_sc_decode_body)


@jax.jit
def kernel(x, W_E, b_E, W_D, b_D):
    b_E2 = b_E.reshape(1, D_SAE_)

    latents_masked, flags, scale_blk = pl.pallas_call(
        _enc_topk_body,
        grid=(N_TILES_,),
        in_specs=[
            pl.BlockSpec((N_TOK_, D_MODEL_), lambda i: (0, 0)),
            pl.BlockSpec((D_MODEL_, TS_), lambda i: (0, i)),
            pl.BlockSpec((1, TS_), lambda i: (0, i)),
        ],
        out_specs=[
            pl.BlockSpec((N_TOK_, D_SAE_), lambda i: (0, 0)),
            pl.BlockSpec((N_TOK_, N_REGIONS_), lambda i: (0, 0)),
            pl.BlockSpec((N_TOK_, 128), lambda i: (0, 0)),
        ],
        out_shape=[
            jax.ShapeDtypeStruct((N_TOK_, D_SAE_), jnp.float32),
            jax.ShapeDtypeStruct((N_TOK_, N_REGIONS_), jnp.float32),
            jax.ShapeDtypeStruct((N_TOK_, 128), jnp.float32),
        ],
        scratch_shapes=[
            pltpu.VMEM((N_TOK_, D_SAE_), jnp.float32),
            pltpu.VMEM((N_TOK_, 1), jnp.float32),
        ],
        compiler_params=pltpu.CompilerParams(
            dimension_semantics=("arbitrary",)),
    )(x, W_E, b_E2)

    scale_vec = scale_blk[:, 0]
    recon = _sc_decode(latents_masked, flags, W_D, b_D, scale_vec)
    return recon, latents_masked
